# baseline (device time: 14132 ns/iter reference)
import jax
import jax.numpy as jnp
from jax import lax
from jax.experimental import pallas as pl
from jax.experimental.pallas import tpu as pltpu

N_DEV = 8
BLOCK = 4


def kernel(A, B):
    m, k = A.shape
    _, n = B.shape
    m_out = m // N_DEV

    def body(
        a_hbm, b_hbm, out_ref,
        a_ref, b_ref, a16_ref, b16_ref, send_ref, recv_ref,
        load_sems, send_sems, recv_sems,
    ):
        p = lax.axis_index("i")

        load_a = pltpu.make_async_copy(a_hbm, a_ref, load_sems.at[0])
        load_b = pltpu.make_async_copy(b_hbm, b_ref, load_sems.at[1])
        load_a.start()
        load_b.start()

        barrier_sem = pltpu.get_barrier_semaphore()
        for off in range(1, N_DEV):
            other = lax.rem(p + off, N_DEV)
            pl.semaphore_signal(
                barrier_sem, inc=1,
                device_id=(other,), device_id_type=pl.DeviceIdType.MESH,
            )

        load_b.wait()
        b16_ref[...] = b_ref[...].astype(jnp.bfloat16)
        load_a.wait()
        for off in range(1, N_DEV + 1):
            dest = lax.rem(p + off, N_DEV)
            a16_ref[pl.ds((off - 1) * m_out, m_out), :] = (
                a_ref[pl.ds(dest * m_out, m_out), :].astype(jnp.bfloat16)
            )

        col = lax.broadcasted_iota(jnp.int32, (1, n), 1)
        rdmas = []
        for blk in range(0, N_DEV, BLOCK):
            rows = pl.ds(blk * m_out, BLOCK * m_out)
            part = jnp.dot(
                a16_ref[rows, :], b16_ref[...],
                preferred_element_type=jnp.float32,
            )

            if blk + BLOCK >= N_DEV:
                own_part = part

            s = jnp.maximum(jnp.max(jnp.abs(part)), 1e-20) / 127.0
            e = jnp.floor(jnp.log2(s))
            mant = jnp.round(s * jnp.exp2(-e) * 64.0)
            e = jnp.where(mant >= 128.0, e + 1.0, e)
            mant = jnp.where(mant >= 128.0, 64.0, mant)
            s_rec = mant * (1.0 / 64.0) * jnp.exp2(e)
            q = jnp.clip(
                jnp.round(part * (1.0 / s_rec)), -127.0, 127.0
            ).astype(jnp.int8)
            scale_row = jnp.where(
                col < (n // 2), mant.astype(jnp.int32), e.astype(jnp.int32)
            ).astype(jnp.int8)

            if blk == 0:
                pl.semaphore_wait(barrier_sem, N_DEV - 1)
            for r in range(blk, blk + BLOCK):
                off = r + 1
                if off >= N_DEV:
                    continue
                dest = lax.rem(p + off, N_DEV)
                send_ref[off, 0:m_out, :] = q[(r - blk) * m_out:(r - blk + 1) * m_out, :]
                send_ref[off, m_out:m_out + 1, :] = scale_row
                rdma = pltpu.make_async_remote_copy(
                    src_ref=send_ref.at[off],
                    dst_ref=recv_ref.at[off],
                    send_sem=send_sems.at[off],
                    recv_sem=recv_sems.at[off],
                    device_id=(dest,),
                    device_id_type=pl.DeviceIdType.MESH,
                )
                rdma.start()
                rdmas.append(rdma)

        out_ref[...] = own_part[(BLOCK - 1) * m_out:, :]

        for off in range(1, N_DEV):
            rdmas[off - 1].wait_recv()
            m_in = recv_ref[off, m_out:m_out + 1, 0:128].astype(jnp.float32)[0, 0]
            e_in = recv_ref[off, m_out:m_out + 1, n - 128:n].astype(jnp.float32)[0, 0]
            s_in = m_in * (1.0 / 64.0) * jnp.exp2(e_in)
            out_ref[...] = out_ref[...] + (
                recv_ref[off, 0:m_out, :].astype(jnp.float32) * s_in
            )

        for rdma in rdmas:
            rdma.wait_send()

    return pl.pallas_call(
        body,
        out_shape=jax.ShapeDtypeStruct((m_out, n), jnp.float32),
        in_specs=[
            pl.BlockSpec(memory_space=pltpu.HBM),
            pl.BlockSpec(memory_space=pltpu.HBM),
        ],
        out_specs=pl.BlockSpec(memory_space=pltpu.VMEM),
        scratch_shapes=[
            pltpu.VMEM((m, k), jnp.float32),
            pltpu.VMEM((k, n), jnp.float32),
            pltpu.VMEM((m, k), jnp.bfloat16),
            pltpu.VMEM((k, n), jnp.bfloat16),
            pltpu.VMEM((N_DEV, m_out + 1, n), jnp.int8),
            pltpu.VMEM((N_DEV, m_out + 1, n), jnp.int8),
            pltpu.SemaphoreType.DMA((2,)),
            pltpu.SemaphoreType.DMA((N_DEV,)),
            pltpu.SemaphoreType.DMA((N_DEV,)),
        ],
        compiler_params=pltpu.CompilerParams(collective_id=0),
    )(A, B)


# device time: 13597 ns/iter; 1.0393x vs baseline; 1.0393x over previous
import jax
import jax.numpy as jnp
from jax import lax
from jax.experimental import pallas as pl
from jax.experimental.pallas import tpu as pltpu

N_DEV = 8
BLOCK = 2


def kernel(A, B):
    m, k = A.shape
    _, n = B.shape
    m_out = m // N_DEV

    def body(
        a_ref, b_ref, out_ref,
        a16_ref, b16_ref, send_ref, recv_ref, send_sems, recv_sems,
    ):
        p = lax.axis_index("i")

        barrier_sem = pltpu.get_barrier_semaphore()
        for off in range(1, N_DEV):
            other = lax.rem(p + off, N_DEV)
            pl.semaphore_signal(
                barrier_sem, inc=1,
                device_id=(other,), device_id_type=pl.DeviceIdType.MESH,
            )

        b16_ref[...] = b_ref[...].astype(jnp.bfloat16)
        for off in range(1, N_DEV + 1):
            dest = lax.rem(p + off, N_DEV)
            a16_ref[pl.ds((off - 1) * m_out, m_out), :] = (
                a_ref[pl.ds(dest * m_out, m_out), :].astype(jnp.bfloat16)
            )

        col = lax.broadcasted_iota(jnp.int32, (1, n), 1)
        rdmas = []
        for blk in range(0, N_DEV, BLOCK):
            rows = pl.ds(blk * m_out, BLOCK * m_out)
            part = jnp.dot(
                a16_ref[rows, :], b16_ref[...],
                preferred_element_type=jnp.float32,
            )

            if blk + BLOCK >= N_DEV:
                own_part = part

            s = jnp.maximum(jnp.max(jnp.abs(part)), 1e-20) / 127.0
            e = jnp.floor(jnp.log2(s))
            mant = jnp.round(s * jnp.exp2(-e) * 64.0)
            e = jnp.where(mant >= 128.0, e + 1.0, e)
            mant = jnp.where(mant >= 128.0, 64.0, mant)
            s_rec = mant * (1.0 / 64.0) * jnp.exp2(e)
            q = jnp.clip(
                jnp.round(part * (1.0 / s_rec)), -127.0, 127.0
            ).astype(jnp.int8)
            scale_row = jnp.where(
                col < (n // 2), mant.astype(jnp.int32), e.astype(jnp.int32)
            ).astype(jnp.int8)

            if blk == 0:
                pl.semaphore_wait(barrier_sem, N_DEV - 1)
            for r in range(blk, blk + BLOCK):
                off = r + 1
                if off >= N_DEV:
                    continue
                dest = lax.rem(p + off, N_DEV)
                send_ref[off, 0:m_out, :] = q[(r - blk) * m_out:(r - blk + 1) * m_out, :]
                send_ref[off, m_out:m_out + 1, :] = scale_row
                rdma = pltpu.make_async_remote_copy(
                    src_ref=send_ref.at[off],
                    dst_ref=recv_ref.at[off],
                    send_sem=send_sems.at[off],
                    recv_sem=recv_sems.at[off],
                    device_id=(dest,),
                    device_id_type=pl.DeviceIdType.MESH,
                )
                rdma.start()
                rdmas.append(rdma)

        out_ref[...] = own_part[(BLOCK - 1) * m_out:, :]

        for off in range(1, N_DEV):
            rdmas[off - 1].wait_recv()
            m_in = recv_ref[off, m_out:m_out + 1, 0:128].astype(jnp.float32)[0, 0]
            e_in = recv_ref[off, m_out:m_out + 1, n - 128:n].astype(jnp.float32)[0, 0]
            s_in = m_in * (1.0 / 64.0) * jnp.exp2(e_in)
            out_ref[...] = out_ref[...] + (
                recv_ref[off, 0:m_out, :].astype(jnp.float32) * s_in
            )

        for rdma in rdmas:
            rdma.wait_send()

    return pl.pallas_call(
        body,
        out_shape=jax.ShapeDtypeStruct((m_out, n), jnp.float32),
        in_specs=[
            pl.BlockSpec(memory_space=pltpu.VMEM),
            pl.BlockSpec(memory_space=pltpu.VMEM),
        ],
        out_specs=pl.BlockSpec(memory_space=pltpu.VMEM),
        scratch_shapes=[
            pltpu.VMEM((m, k), jnp.bfloat16),
            pltpu.VMEM((k, n), jnp.bfloat16),
            pltpu.VMEM((N_DEV, m_out + 1, n), jnp.int8),
            pltpu.VMEM((N_DEV, m_out + 1, n), jnp.int8),
            pltpu.SemaphoreType.DMA((N_DEV,)),
            pltpu.SemaphoreType.DMA((N_DEV,)),
        ],
        compiler_params=pltpu.CompilerParams(collective_id=0),
    )(A, B)


# device time: 13570 ns/iter; 1.0414x vs baseline; 1.0020x over previous
import jax
import jax.numpy as jnp
from jax import lax
from jax.experimental import pallas as pl
from jax.experimental.pallas import tpu as pltpu

N_DEV = 8
BLOCK = 2


def kernel(A, B):
    m, k = A.shape
    _, n = B.shape
    m_out = m // N_DEV

    def body(
        a_ref, b_ref, out_ref,
        a16_ref, b16_ref, send_ref, recv_ref, send_sems, recv_sems,
    ):
        p = lax.axis_index("i")

        barrier_sem = pltpu.get_barrier_semaphore()
        for off in range(1, N_DEV):
            other = lax.rem(p + off, N_DEV)
            pl.semaphore_signal(
                barrier_sem, inc=1,
                device_id=(other,), device_id_type=pl.DeviceIdType.MESH,
            )

        b16_ref[...] = b_ref[...].astype(jnp.bfloat16)
        for off in range(1, N_DEV + 1):
            dest = lax.rem(p + off, N_DEV)
            a16_ref[pl.ds((off - 1) * m_out, m_out), :] = (
                a_ref[pl.ds(dest * m_out, m_out), :].astype(jnp.bfloat16)
            )

        col = lax.broadcasted_iota(jnp.int32, (1, n), 1)
        rdmas = []
        for blk in range(0, N_DEV, BLOCK):
            rows = pl.ds(blk * m_out, BLOCK * m_out)
            part = jnp.dot(
                a16_ref[rows, :], b16_ref[...],
                preferred_element_type=jnp.float32,
            )

            if blk + BLOCK >= N_DEV:
                own_part = part

            s = jnp.maximum(jnp.max(jnp.abs(part)), 1e-20) / 127.0
            e = jnp.floor(jnp.log2(s))
            mant = jnp.ceil(s * jnp.exp2(-e) * 64.0)
            e = jnp.where(mant >= 128.0, e + 1.0, e)
            mant = jnp.where(mant >= 128.0, 64.0, mant)
            s_rec = mant * (1.0 / 64.0) * jnp.exp2(e)
            q = jnp.round(part * (1.0 / s_rec)).astype(jnp.int8)
            scale_row = jnp.where(
                col < (n // 2), mant.astype(jnp.int32), e.astype(jnp.int32)
            ).astype(jnp.int8)

            if blk == 0:
                pl.semaphore_wait(barrier_sem, N_DEV - 1)
            for r in range(blk, blk + BLOCK):
                off = r + 1
                if off >= N_DEV:
                    continue
                dest = lax.rem(p + off, N_DEV)
                send_ref[off, 0:m_out, :] = q[(r - blk) * m_out:(r - blk + 1) * m_out, :]
                send_ref[off, m_out:m_out + 1, :] = scale_row
                rdma = pltpu.make_async_remote_copy(
                    src_ref=send_ref.at[off],
                    dst_ref=recv_ref.at[off],
                    send_sem=send_sems.at[off],
                    recv_sem=recv_sems.at[off],
                    device_id=(dest,),
                    device_id_type=pl.DeviceIdType.MESH,
                )
                rdma.start()
                rdmas.append(rdma)

        out_ref[...] = own_part[(BLOCK - 1) * m_out:, :]

        for off in range(1, N_DEV):
            rdmas[off - 1].wait_recv()
            m_in = recv_ref[off, m_out:m_out + 1, 0:128].astype(jnp.float32)[0, 0]
            e_in = recv_ref[off, m_out:m_out + 1, n - 128:n].astype(jnp.float32)[0, 0]
            s_in = m_in * (1.0 / 64.0) * jnp.exp2(e_in)
            out_ref[...] = out_ref[...] + (
                recv_ref[off, 0:m_out, :].astype(jnp.float32) * s_in
            )

        for rdma in rdmas:
            rdma.wait_send()

    return pl.pallas_call(
        body,
        out_shape=jax.ShapeDtypeStruct((m_out, n), jnp.float32),
        in_specs=[
            pl.BlockSpec(memory_space=pltpu.VMEM),
            pl.BlockSpec(memory_space=pltpu.VMEM),
        ],
        out_specs=pl.BlockSpec(memory_space=pltpu.VMEM),
        scratch_shapes=[
            pltpu.VMEM((m, k), jnp.bfloat16),
            pltpu.VMEM((k, n), jnp.bfloat16),
            pltpu.VMEM((N_DEV, m_out + 1, n), jnp.int8),
            pltpu.VMEM((N_DEV, m_out + 1, n), jnp.int8),
            pltpu.SemaphoreType.DMA((N_DEV,)),
            pltpu.SemaphoreType.DMA((N_DEV,)),
        ],
        compiler_params=pltpu.CompilerParams(collective_id=0),
    )(A, B)
